# baseline (speedup 1.0000x reference)
import functools

import jax
import jax.numpy as jnp
from jax import lax
from jax.experimental import pallas as pl
from jax.experimental.pallas import tpu as pltpu
from jax.experimental.pallas import tpu_sc as plsc


def _sc_gather(nfeat, src, dst):
    n, d = nfeat.shape
    e = src.shape[0]
    info = plsc.get_sparse_core_info()
    nw = info.num_cores * info.num_subcores
    per_w = e // nw
    ch = 400
    n_ch = per_w // ch
    assert per_w * nw == e and n_ch * ch == per_w

    mesh = plsc.VectorSubcoreMesh(core_axis_name="c", subcore_axis_name="s")

    @functools.partial(
        pl.kernel,
        out_type=(
            jax.ShapeDtypeStruct((e, d), jnp.float32),
            jax.ShapeDtypeStruct((e, d), jnp.float32),
        ),
        mesh=mesh,
        scratch_types=[
            pltpu.VMEM((per_w,), jnp.int32),
            pltpu.VMEM((per_w,), jnp.int32),
            pltpu.VMEM((ch, d), jnp.float32),
            pltpu.VMEM((ch, d), jnp.float32),
            pltpu.SemaphoreType.DMA,
            pltpu.SemaphoreType.DMA,
        ],
    )
    def gather_kernel(nfeat_hbm, src_hbm, dst_hbm, gsrc_hbm, gdst_hbm,
                      src_v, dst_v, rows_s, rows_d, sem_s, sem_d):
        wid = lax.axis_index("s") * info.num_cores + lax.axis_index("c")
        base = pl.multiple_of(wid * per_w, ch)
        pltpu.sync_copy(src_hbm.at[pl.ds(base, per_w)], src_v)
        pltpu.sync_copy(dst_hbm.at[pl.ds(base, per_w)], dst_v)

        def body(j, carry):
            off = pl.multiple_of(j * ch, ch)
            cp_s = pltpu.async_copy(
                nfeat_hbm.at[src_v.at[pl.ds(off, ch)]], rows_s, sem_s)
            cp_d = pltpu.async_copy(
                nfeat_hbm.at[dst_v.at[pl.ds(off, ch)]], rows_d, sem_d)
            cp_s.wait()
            pltpu.sync_copy(rows_s, gsrc_hbm.at[pl.ds(base + off, ch)])
            cp_d.wait()
            pltpu.sync_copy(rows_d, gdst_hbm.at[pl.ds(base + off, ch)])
            return carry

        lax.fori_loop(0, n_ch, body, 0)

    return gather_kernel(nfeat, src, dst)


def _tc_body(e_ref, s_ref, d_ref, we_ref, ws_ref, wd_ref, bin_ref,
             wo_ref, bo_ref, g_ref, b_ref, o_ref):
    bf = jnp.bfloat16
    f32 = jnp.float32
    x = (jnp.dot(e_ref[...].astype(bf), we_ref[...].astype(bf),
                 preferred_element_type=f32)
         + jnp.dot(s_ref[...].astype(bf), ws_ref[...].astype(bf),
                   preferred_element_type=f32)
         + jnp.dot(d_ref[...].astype(bf), wd_ref[...].astype(bf),
                   preferred_element_type=f32)
         + bin_ref[...])
    h = x / (1.0 + jnp.exp(-x))
    out = jnp.dot(h.astype(bf), wo_ref[...].astype(bf),
                  preferred_element_type=f32) + bo_ref[...]
    mean = jnp.mean(out, axis=-1, keepdims=True)
    var = jnp.mean((out - mean) ** 2, axis=-1, keepdims=True)
    o_ref[...] = (out - mean) * lax.rsqrt(var + 1e-5) * g_ref[...] + b_ref[...]


def _tc_fused(efeat, gsrc, gdst, we_t, ws_t, wd_t, b_in, wo_t, b_out,
              ln_gamma, ln_beta, block_e):
    e, d = efeat.shape
    h = we_t.shape[1]
    o = wo_t.shape[1]
    grid = (e // block_e,)
    row_spec = pl.BlockSpec((block_e, d), lambda i: (i, 0))
    full = lambda r, c: pl.BlockSpec((r, c), lambda i: (0, 0))
    return pl.pallas_call(
        _tc_body,
        grid=grid,
        in_specs=[
            row_spec, row_spec, row_spec,
            full(d, h), full(d, h), full(d, h), full(1, h),
            full(h, o), full(1, o), full(1, o), full(1, o),
        ],
        out_specs=pl.BlockSpec((block_e, o), lambda i: (i, 0)),
        out_shape=jax.ShapeDtypeStruct((e, o), jnp.float32),
        compiler_params=pltpu.CompilerParams(
            dimension_semantics=("arbitrary",)),
    )(efeat, gsrc, gdst, we_t, ws_t, wd_t, b_in, wo_t, b_out,
      ln_gamma, ln_beta)


def kernel(efeat, nfeat, edge_index, W_efeat, W_src, W_dst, b_in, W_out,
           b_out, ln_gamma, ln_beta):
    h = W_efeat.shape[0]
    o = W_out.shape[0]
    gsrc, gdst = _sc_gather(nfeat, edge_index[0], edge_index[1])
    return _tc_fused(
        efeat, gsrc, gdst,
        W_efeat.T, W_src.T, W_dst.T, b_in.reshape(1, h),
        W_out.T, b_out.reshape(1, o),
        ln_gamma.reshape(1, o), ln_beta.reshape(1, o),
        block_e=2560,
    )


# speedup vs baseline: 1.1385x; 1.1385x over previous
import functools

import jax
import jax.numpy as jnp
from jax import lax
from jax.experimental import pallas as pl
from jax.experimental.pallas import tpu as pltpu
from jax.experimental.pallas import tpu_sc as plsc


def _sc_gather(nfeat, src, dst, ch):
    n, d = nfeat.shape
    e = src.shape[0]
    info = plsc.get_sparse_core_info()
    nw = info.num_cores * info.num_subcores
    per_w = e // nw
    n_ch = per_w // ch
    assert per_w * nw == e and n_ch * ch == per_w and ch % 8 == 0

    mesh = plsc.VectorSubcoreMesh(core_axis_name="c", subcore_axis_name="s")

    @functools.partial(
        pl.kernel,
        out_type=(
            jax.ShapeDtypeStruct((e, d), jnp.float32),
            jax.ShapeDtypeStruct((e, d), jnp.float32),
        ),
        mesh=mesh,
        scratch_types=[
            pltpu.VMEM((per_w,), jnp.int32),
            pltpu.VMEM((per_w,), jnp.int32),
            pltpu.VMEM((ch, d), jnp.float32),
            pltpu.VMEM((ch, d), jnp.float32),
            pltpu.SemaphoreType.DMA,
            pltpu.SemaphoreType.DMA,
        ],
    )
    def gather_kernel(nfeat_hbm, src_hbm, dst_hbm, gsrc_hbm, gdst_hbm,
                      src_v, dst_v, rows_s, rows_d, sem_s, sem_d):
        wid = lax.axis_index("s") * info.num_cores + lax.axis_index("c")
        base = pl.multiple_of(wid * per_w, ch)
        pltpu.sync_copy(src_hbm.at[pl.ds(base, per_w)], src_v)
        pltpu.sync_copy(dst_hbm.at[pl.ds(base, per_w)], dst_v)

        def body(j, carry):
            off = pl.multiple_of(j * ch, ch)
            cp_s = pltpu.async_copy(
                nfeat_hbm.at[src_v.at[pl.ds(off, ch)]], rows_s, sem_s)
            cp_d = pltpu.async_copy(
                nfeat_hbm.at[dst_v.at[pl.ds(off, ch)]], rows_d, sem_d)
            cp_s.wait()
            pltpu.sync_copy(rows_s, gsrc_hbm.at[pl.ds(base + off, ch)])
            cp_d.wait()
            pltpu.sync_copy(rows_d, gdst_hbm.at[pl.ds(base + off, ch)])
            return carry

        lax.fori_loop(0, n_ch, body, 0)

    return gather_kernel(nfeat, src, dst)


def _tc_body(e_ref, s_ref, d_ref, we_ref, ws_ref, wd_ref, bin_ref,
             wo_ref, bo_ref, g_ref, b_ref, *rest):
    o_ref = rest[-1]
    bf = jnp.bfloat16
    f32 = jnp.float32
    x = (jnp.dot(e_ref[...].astype(bf), we_ref[...].astype(bf),
                 preferred_element_type=f32)
         + jnp.dot(s_ref[...].astype(bf), ws_ref[...].astype(bf),
                   preferred_element_type=f32)
         + jnp.dot(d_ref[...].astype(bf), wd_ref[...].astype(bf),
                   preferred_element_type=f32)
         + bin_ref[...])
    h = x / (1.0 + jnp.exp(-x))
    out = jnp.dot(h.astype(bf), wo_ref[...].astype(bf),
                  preferred_element_type=f32) + bo_ref[...]
    mean = jnp.mean(out, axis=-1, keepdims=True)
    var = jnp.mean((out - mean) ** 2, axis=-1, keepdims=True)
    o_ref[...] = (out - mean) * lax.rsqrt(var + 1e-5) * g_ref[...] + b_ref[...]


def _tc_chunk(y_prev, efeat, gsrc_k, gdst_k, we_t, ws_t, wd_t, b_in, wo_t,
              b_out, ln_gamma, ln_beta, block_e, base_blk):
    e, d = efeat.shape
    h = we_t.shape[1]
    o = wo_t.shape[1]
    ce = gsrc_k.shape[0]
    grid = (ce // block_e,)
    edge_spec = pl.BlockSpec((block_e, d), lambda j: (base_blk + j, 0))
    chunk_spec = pl.BlockSpec((block_e, d), lambda j: (j, 0))
    full = lambda r, c: pl.BlockSpec((r, c), lambda j: (0, 0))
    in_specs = [
        edge_spec, chunk_spec, chunk_spec,
        full(d, h), full(d, h), full(d, h), full(1, h),
        full(h, o), full(1, o), full(1, o), full(1, o),
    ]
    args = [efeat, gsrc_k, gdst_k, we_t, ws_t, wd_t, b_in, wo_t, b_out,
            ln_gamma, ln_beta]
    aliases = {}
    if y_prev is not None:
        in_specs.append(pl.BlockSpec(memory_space=pl.ANY))
        args.append(y_prev)
        aliases = {len(args) - 1: 0}
    return pl.pallas_call(
        _tc_body,
        grid=grid,
        in_specs=in_specs,
        out_specs=pl.BlockSpec((block_e, o), lambda j: (base_blk + j, 0)),
        out_shape=jax.ShapeDtypeStruct((e, o), jnp.float32),
        input_output_aliases=aliases,
        compiler_params=pltpu.CompilerParams(
            dimension_semantics=("arbitrary",)),
    )(*args)


def kernel(efeat, nfeat, edge_index, W_efeat, W_src, W_dst, b_in, W_out,
           b_out, ln_gamma, ln_beta):
    e = efeat.shape[0]
    h = W_efeat.shape[0]
    o = W_out.shape[0]
    n_chunks = 5
    block_e = 2560
    ce = e // n_chunks
    assert ce % block_e == 0

    src = edge_index[0]
    dst = edge_index[1]
    gathered = [
        _sc_gather(nfeat, src[k * ce:(k + 1) * ce], dst[k * ce:(k + 1) * ce],
                   ch=400)
        for k in range(n_chunks)
    ]

    we_t = W_efeat.T
    ws_t = W_src.T
    wd_t = W_dst.T
    wo_t = W_out.T
    b_in2 = b_in.reshape(1, h)
    b_out2 = b_out.reshape(1, o)
    g2 = ln_gamma.reshape(1, o)
    be2 = ln_beta.reshape(1, o)

    y = None
    for k in range(n_chunks):
        gsrc_k, gdst_k = gathered[k]
        y = _tc_chunk(y, efeat, gsrc_k, gdst_k, we_t, ws_t, wd_t, b_in2,
                      wo_t, b_out2, g2, be2, block_e,
                      base_blk=k * (ce // block_e))
    return y


# speedup vs baseline: 1.1735x; 1.0308x over previous
import functools

import jax
import jax.numpy as jnp
from jax import lax
from jax.experimental import pallas as pl
from jax.experimental.pallas import tpu as pltpu
from jax.experimental.pallas import tpu_sc as plsc


def _sc_gather(tab_src, tab_dst, src, dst, ch):
    n, d2 = tab_src.shape
    e = src.shape[0]
    info = plsc.get_sparse_core_info()
    nw = info.num_cores * info.num_subcores
    per_w = e // nw
    n_ch = per_w // ch
    assert per_w * nw == e and n_ch * ch == per_w and ch % 8 == 0

    mesh = plsc.VectorSubcoreMesh(core_axis_name="c", subcore_axis_name="s")

    @functools.partial(
        pl.kernel,
        out_type=jax.ShapeDtypeStruct((e, d2), jnp.int32),
        mesh=mesh,
        scratch_types=[
            pltpu.VMEM((per_w,), jnp.int32),
            pltpu.VMEM((per_w,), jnp.int32),
            pltpu.VMEM((ch, d2), jnp.int32),
            pltpu.VMEM((ch, d2), jnp.int32),
            pltpu.SemaphoreType.DMA,
            pltpu.SemaphoreType.DMA,
        ],
    )
    def gather_kernel(tabs_hbm, tabd_hbm, src_hbm, dst_hbm, g_hbm,
                      src_v, dst_v, rows_a, rows_b, sem_a, sem_b):
        wid = lax.axis_index("s") * info.num_cores + lax.axis_index("c")
        base = pl.multiple_of(wid * per_w, ch)
        pltpu.sync_copy(src_hbm.at[pl.ds(base, per_w)], src_v)
        pltpu.sync_copy(dst_hbm.at[pl.ds(base, per_w)], dst_v)

        def chunk(j, rows, sem):
            off = pl.multiple_of(j * ch, ch)
            pltpu.async_copy(
                tabs_hbm.at[src_v.at[pl.ds(off, ch)]], rows, sem).wait()
            pltpu.async_copy(
                tabd_hbm.at[dst_v.at[pl.ds(off, ch)]], rows, sem,
                add=True).wait()
            pltpu.sync_copy(rows, g_hbm.at[pl.ds(base + off, ch)])

        def body(j2, carry):
            chunk(j2 * 2, rows_a, sem_a)
            chunk(j2 * 2 + 1, rows_b, sem_b)
            return carry

        assert n_ch % 2 == 0
        lax.fori_loop(0, n_ch // 2, body, 0)

    return gather_kernel(tab_src, tab_dst, src, dst)


def _tc_body(e_ref, g_ref, we_ref, wa_ref, wb_ref, bin_ref,
             wo_ref, bo_ref, gam_ref, bet_ref, *rest):
    o_ref = rest[-1]
    bf = jnp.bfloat16
    f32 = jnp.float32
    w = g_ref[...].astype(jnp.uint32)
    a = lax.bitcast_convert_type(w << 16, f32).astype(bf)
    b = lax.bitcast_convert_type(w & jnp.uint32(0xFFFF0000), f32).astype(bf)
    x = (jnp.dot(e_ref[...].astype(bf), we_ref[...],
                 preferred_element_type=f32)
         + jnp.dot(a, wa_ref[...], preferred_element_type=f32)
         + jnp.dot(b, wb_ref[...], preferred_element_type=f32)
         + bin_ref[...])
    h = x / (1.0 + jnp.exp(-x))
    out = jnp.dot(h.astype(bf), wo_ref[...],
                  preferred_element_type=f32) + bo_ref[...]
    mean = jnp.mean(out, axis=-1, keepdims=True)
    var = jnp.mean((out - mean) ** 2, axis=-1, keepdims=True)
    o_ref[...] = ((out - mean) * lax.rsqrt(var + 1e-5) * gam_ref[...]
                  + bet_ref[...])


def _tc_chunk(y_prev, efeat, g_k, we_t, wa, wb, b_in, wo_t,
              b_out, ln_gamma, ln_beta, block_e, base_blk):
    e, d = efeat.shape
    h = we_t.shape[1]
    o = wo_t.shape[1]
    ce = g_k.shape[0]
    d2 = g_k.shape[1]
    grid = (ce // block_e,)
    edge_spec = pl.BlockSpec((block_e, d), lambda j: (base_blk + j, 0))
    chunk_spec = pl.BlockSpec((block_e, d2), lambda j: (j, 0))
    full = lambda r, c: pl.BlockSpec((r, c), lambda j: (0, 0))
    in_specs = [
        edge_spec, chunk_spec,
        full(d, h), full(d2, h), full(d2, h), full(1, h),
        full(h, o), full(1, o), full(1, o), full(1, o),
    ]
    args = [efeat, g_k, we_t, wa, wb, b_in, wo_t, b_out, ln_gamma, ln_beta]
    aliases = {}
    if y_prev is not None:
        in_specs.append(pl.BlockSpec(memory_space=pl.ANY))
        args.append(y_prev)
        aliases = {len(args) - 1: 0}
    return pl.pallas_call(
        _tc_body,
        grid=grid,
        in_specs=in_specs,
        out_specs=pl.BlockSpec((block_e, o), lambda j: (base_blk + j, 0)),
        out_shape=jax.ShapeDtypeStruct((e, o), jnp.float32),
        input_output_aliases=aliases,
        compiler_params=pltpu.CompilerParams(
            dimension_semantics=("arbitrary",)),
    )(*args)


def kernel(efeat, nfeat, edge_index, W_efeat, W_src, W_dst, b_in, W_out,
           b_out, ln_gamma, ln_beta):
    n, d = nfeat.shape
    e = efeat.shape[0]
    h = W_efeat.shape[0]
    o = W_out.shape[0]
    n_chunks = 5
    block_e = 2560
    ce = e // n_chunks
    assert ce % block_e == 0

    packed = lax.bitcast_convert_type(
        nfeat.astype(jnp.bfloat16).reshape(n, d // 2, 2), jnp.int32)
    zeros = jnp.zeros_like(packed)
    tab_src = jnp.concatenate([packed, zeros], axis=1)
    tab_dst = jnp.concatenate([zeros, packed], axis=1)

    src = edge_index[0]
    dst = edge_index[1]
    gathered = [
        _sc_gather(tab_src, tab_dst, src[k * ce:(k + 1) * ce],
                   dst[k * ce:(k + 1) * ce], ch=200)
        for k in range(n_chunks)
    ]

    bf = jnp.bfloat16
    we_t = W_efeat.T.astype(bf)
    ws_t = W_src.T
    wd_t = W_dst.T
    wa = jnp.concatenate([ws_t[0::2], wd_t[0::2]], axis=0).astype(bf)
    wb = jnp.concatenate([ws_t[1::2], wd_t[1::2]], axis=0).astype(bf)
    wo_t = W_out.T.astype(bf)
    b_in2 = b_in.reshape(1, h)
    b_out2 = b_out.reshape(1, o)
    g2 = ln_gamma.reshape(1, o)
    be2 = ln_beta.reshape(1, o)

    y = None
    for k in range(n_chunks):
        y = _tc_chunk(y, efeat, gathered[k], we_t, wa, wb, b_in2,
                      wo_t, b_out2, g2, be2, block_e,
                      base_blk=k * (ce // block_e))
    return y


# speedup vs baseline: 1.2565x; 1.0707x over previous
import functools

import jax
import jax.numpy as jnp
from jax import lax
from jax.experimental import pallas as pl
from jax.experimental.pallas import tpu as pltpu
from jax.experimental.pallas import tpu_sc as plsc


def _sc_gather(tab_src, tab_dst, src, dst, ch):
    n, d2 = tab_src.shape
    e = src.shape[0]
    info = plsc.get_sparse_core_info()
    nw = info.num_cores * info.num_subcores
    per_w = e // nw
    n_ch = per_w // ch
    assert per_w * nw == e and n_ch * ch == per_w and ch % 8 == 0

    mesh = plsc.VectorSubcoreMesh(core_axis_name="c", subcore_axis_name="s")

    @functools.partial(
        pl.kernel,
        out_type=jax.ShapeDtypeStruct((e, d2), jnp.int32),
        mesh=mesh,
        scratch_types=[
            pltpu.VMEM((per_w,), jnp.int32),
            pltpu.VMEM((per_w,), jnp.int32),
            pltpu.VMEM((ch, d2), jnp.int32),
            pltpu.VMEM((ch, d2), jnp.int32),
            pltpu.SemaphoreType.DMA,
            pltpu.SemaphoreType.DMA,
        ],
    )
    def gather_kernel(tabs_hbm, tabd_hbm, src_hbm, dst_hbm, g_hbm,
                      src_v, dst_v, rows_a, rows_b, sem_a, sem_b):
        wid = lax.axis_index("s") * info.num_cores + lax.axis_index("c")
        base = pl.multiple_of(wid * per_w, ch)
        pltpu.sync_copy(src_hbm.at[pl.ds(base, per_w)], src_v)
        pltpu.sync_copy(dst_hbm.at[pl.ds(base, per_w)], dst_v)

        def chunk(j, rows, sem):
            off = pl.multiple_of(j * ch, ch)
            pltpu.async_copy(
                tabs_hbm.at[src_v.at[pl.ds(off, ch)]], rows, sem).wait()
            pltpu.async_copy(
                tabd_hbm.at[dst_v.at[pl.ds(off, ch)]], rows, sem,
                add=True).wait()
            pltpu.sync_copy(rows, g_hbm.at[pl.ds(base + off, ch)])

        def body(j2, carry):
            chunk(j2 * 2, rows_a, sem_a)
            chunk(j2 * 2 + 1, rows_b, sem_b)
            return carry

        assert n_ch % 2 == 0
        lax.fori_loop(0, n_ch // 2, body, 0)

    return gather_kernel(tab_src, tab_dst, src, dst)


def _tc_body(e_ref, g_ref, wsd_ref, bin_ref,
             wo_ref, bo_ref, gam_ref, bet_ref, *rest):
    o_ref = rest[-1]
    bf = jnp.bfloat16
    f32 = jnp.float32
    w = g_ref[...].astype(jnp.uint32)
    a = lax.bitcast_convert_type(w << 16, f32).astype(bf)
    b = lax.bitcast_convert_type(w & jnp.uint32(0xFFFF0000), f32).astype(bf)
    xin = jnp.concatenate([e_ref[...].astype(bf), a, b], axis=1)
    x = (jnp.dot(xin, wsd_ref[...], preferred_element_type=f32)
         + bin_ref[...])
    h = x / (1.0 + jnp.exp(-x))
    out = jnp.dot(h.astype(bf), wo_ref[...],
                  preferred_element_type=f32) + bo_ref[...]
    mean = jnp.mean(out, axis=-1, keepdims=True)
    var = jnp.mean((out - mean) ** 2, axis=-1, keepdims=True)
    o_ref[...] = ((out - mean) * lax.rsqrt(var + 1e-5) * gam_ref[...]
                  + bet_ref[...])


def _tc_chunk(y_prev, efeat, g_k, wsd, b_in, wo_t,
              b_out, ln_gamma, ln_beta, block_e, base_blk):
    e, d = efeat.shape
    h = wsd.shape[1]
    o = wo_t.shape[1]
    ce = g_k.shape[0]
    d2 = g_k.shape[1]
    grid = (ce // block_e,)
    edge_spec = pl.BlockSpec((block_e, d), lambda j: (base_blk + j, 0))
    chunk_spec = pl.BlockSpec((block_e, d2), lambda j: (j, 0))
    full = lambda r, c: pl.BlockSpec((r, c), lambda j: (0, 0))
    in_specs = [
        edge_spec, chunk_spec,
        full(d + 2 * d2, h), full(1, h),
        full(h, o), full(1, o), full(1, o), full(1, o),
    ]
    args = [efeat, g_k, wsd, b_in, wo_t, b_out, ln_gamma, ln_beta]
    aliases = {}
    if y_prev is not None:
        in_specs.append(pl.BlockSpec(memory_space=pl.ANY))
        args.append(y_prev)
        aliases = {len(args) - 1: 0}
    return pl.pallas_call(
        _tc_body,
        grid=grid,
        in_specs=in_specs,
        out_specs=pl.BlockSpec((block_e, o), lambda j: (base_blk + j, 0)),
        out_shape=jax.ShapeDtypeStruct((e, o), jnp.float32),
        input_output_aliases=aliases,
        compiler_params=pltpu.CompilerParams(
            dimension_semantics=("arbitrary",)),
    )(*args)


def kernel(efeat, nfeat, edge_index, W_efeat, W_src, W_dst, b_in, W_out,
           b_out, ln_gamma, ln_beta):
    n, d = nfeat.shape
    e = efeat.shape[0]
    h = W_efeat.shape[0]
    o = W_out.shape[0]
    n_chunks = 5
    block_e = 2560
    ce = e // n_chunks
    assert ce % block_e == 0

    packed = lax.bitcast_convert_type(
        nfeat.astype(jnp.bfloat16).reshape(n, d // 2, 2), jnp.int32)
    zeros = jnp.zeros_like(packed)
    tab_src = jnp.concatenate([packed, zeros], axis=1)
    tab_dst = jnp.concatenate([zeros, packed], axis=1)

    src = edge_index[0]
    dst = edge_index[1]
    gathered = [
        _sc_gather(tab_src, tab_dst, src[k * ce:(k + 1) * ce],
                   dst[k * ce:(k + 1) * ce], ch=200)
        for k in range(n_chunks)
    ]

    bf = jnp.bfloat16
    ws_t = W_src.T
    wd_t = W_dst.T
    wsd = jnp.concatenate(
        [W_efeat.T, ws_t[0::2], wd_t[0::2], ws_t[1::2], wd_t[1::2]],
        axis=0).astype(bf)
    wo_t = W_out.T.astype(bf)
    b_in2 = b_in.reshape(1, h)
    b_out2 = b_out.reshape(1, o)
    g2 = ln_gamma.reshape(1, o)
    be2 = ln_beta.reshape(1, o)

    y = None
    for k in range(n_chunks):
        y = _tc_chunk(y, efeat, gathered[k], wsd, b_in2,
                      wo_t, b_out2, g2, be2, block_e,
                      base_blk=k * (ce // block_e))
    return y


# speedup vs baseline: 1.2640x; 1.0060x over previous
import functools

import jax
import jax.numpy as jnp
from jax import lax
from jax.experimental import pallas as pl
from jax.experimental.pallas import tpu as pltpu
from jax.experimental.pallas import tpu_sc as plsc


def _sc_gather(tab_src, tab_dst, src, dst, ch):
    n, d2 = tab_src.shape
    e = src.shape[0]
    info = plsc.get_sparse_core_info()
    nw = info.num_cores * info.num_subcores
    per_w = e // nw
    n_ch = per_w // ch
    assert per_w * nw == e and n_ch * ch == per_w and ch % 8 == 0

    mesh = plsc.VectorSubcoreMesh(core_axis_name="c", subcore_axis_name="s")

    @functools.partial(
        pl.kernel,
        out_type=jax.ShapeDtypeStruct((e, d2), jnp.int32),
        mesh=mesh,
        scratch_types=[
            pltpu.VMEM((per_w,), jnp.int32),
            pltpu.VMEM((per_w,), jnp.int32),
            pltpu.VMEM((ch, d2), jnp.int32),
            pltpu.VMEM((ch, d2), jnp.int32),
            pltpu.SemaphoreType.DMA,
            pltpu.SemaphoreType.DMA,
        ],
    )
    def gather_kernel(tabs_hbm, tabd_hbm, src_hbm, dst_hbm, g_hbm,
                      src_v, dst_v, rows_a, rows_b, sem_a, sem_b):
        wid = lax.axis_index("s") * info.num_cores + lax.axis_index("c")
        base = pl.multiple_of(wid * per_w, ch)
        pltpu.sync_copy(src_hbm.at[pl.ds(base, per_w)], src_v)
        pltpu.sync_copy(dst_hbm.at[pl.ds(base, per_w)], dst_v)

        def chunk(j, rows, sem):
            off = pl.multiple_of(j * ch, ch)
            pltpu.async_copy(
                tabs_hbm.at[src_v.at[pl.ds(off, ch)]], rows, sem).wait()
            pltpu.async_copy(
                tabd_hbm.at[dst_v.at[pl.ds(off, ch)]], rows, sem,
                add=True).wait()
            pltpu.sync_copy(rows, g_hbm.at[pl.ds(base + off, ch)])

        def body(j2, carry):
            chunk(j2 * 2, rows_a, sem_a)
            chunk(j2 * 2 + 1, rows_b, sem_b)
            return carry

        assert n_ch % 2 == 0
        lax.fori_loop(0, n_ch // 2, body, 0)

    return gather_kernel(tab_src, tab_dst, src, dst)


def _tc_body(e_ref, g_ref, wsd_ref, bin_ref,
             wo_ref, bo_ref, gam_ref, bet_ref, *rest):
    o_ref = rest[-1]
    bf = jnp.bfloat16
    f32 = jnp.float32
    w = g_ref[...].astype(jnp.uint32)
    a = lax.bitcast_convert_type(w << 16, f32).astype(bf)
    b = lax.bitcast_convert_type(w & jnp.uint32(0xFFFF0000), f32).astype(bf)
    xin = jnp.concatenate([e_ref[...].astype(bf), a, b], axis=1)
    x = (jnp.dot(xin, wsd_ref[...], preferred_element_type=f32)
         + bin_ref[...])
    h = x / (1.0 + jnp.exp(-x))
    out = jnp.dot(h.astype(bf), wo_ref[...],
                  preferred_element_type=f32) + bo_ref[...]
    mean = jnp.mean(out, axis=-1, keepdims=True)
    var = jnp.mean((out - mean) ** 2, axis=-1, keepdims=True)
    o_ref[...] = ((out - mean) * lax.rsqrt(var + 1e-5) * gam_ref[...]
                  + bet_ref[...])


def _tc_chunk(y_prev, efeat, g_k, wsd, b_in, wo_t,
              b_out, ln_gamma, ln_beta, block_e, base_blk):
    e, d = efeat.shape
    h = wsd.shape[1]
    o = wo_t.shape[1]
    ce = g_k.shape[0]
    d2 = g_k.shape[1]
    grid = (ce // block_e,)
    edge_spec = pl.BlockSpec((block_e, d), lambda j: (base_blk + j, 0))
    chunk_spec = pl.BlockSpec((block_e, d2), lambda j: (j, 0))
    full = lambda r, c: pl.BlockSpec((r, c), lambda j: (0, 0))
    in_specs = [
        edge_spec, chunk_spec,
        full(d + 2 * d2, h), full(1, h),
        full(h, o), full(1, o), full(1, o), full(1, o),
    ]
    args = [efeat, g_k, wsd, b_in, wo_t, b_out, ln_gamma, ln_beta]
    aliases = {}
    if y_prev is not None:
        in_specs.append(pl.BlockSpec(memory_space=pl.ANY))
        args.append(y_prev)
        aliases = {len(args) - 1: 0}
    return pl.pallas_call(
        _tc_body,
        grid=grid,
        in_specs=in_specs,
        out_specs=pl.BlockSpec((block_e, o), lambda j: (base_blk + j, 0)),
        out_shape=jax.ShapeDtypeStruct((e, o), jnp.float32),
        input_output_aliases=aliases,
        compiler_params=pltpu.CompilerParams(
            dimension_semantics=("arbitrary",)),
    )(*args)


def kernel(efeat, nfeat, edge_index, W_efeat, W_src, W_dst, b_in, W_out,
           b_out, ln_gamma, ln_beta):
    n, d = nfeat.shape
    e = efeat.shape[0]
    h = W_efeat.shape[0]
    o = W_out.shape[0]
    block_e = 2560
    sizes = [12800, 76800, 76800, 76800, 76800]
    assert sum(sizes) == e and all(s % block_e == 0 for s in sizes)
    bounds = [0]
    for s in sizes:
        bounds.append(bounds[-1] + s)

    packed = lax.bitcast_convert_type(
        nfeat.astype(jnp.bfloat16).reshape(n, d // 2, 2), jnp.int32)
    zeros = jnp.zeros_like(packed)
    tab_src = jnp.concatenate([packed, zeros], axis=1)
    tab_dst = jnp.concatenate([zeros, packed], axis=1)

    src = edge_index[0]
    dst = edge_index[1]
    gathered = [
        _sc_gather(tab_src, tab_dst, src[bounds[k]:bounds[k + 1]],
                   dst[bounds[k]:bounds[k + 1]], ch=200)
        for k in range(len(sizes))
    ]

    bf = jnp.bfloat16
    ws_t = W_src.T
    wd_t = W_dst.T
    wsd = jnp.concatenate(
        [W_efeat.T, ws_t[0::2], wd_t[0::2], ws_t[1::2], wd_t[1::2]],
        axis=0).astype(bf)
    wo_t = W_out.T.astype(bf)
    b_in2 = b_in.reshape(1, h)
    b_out2 = b_out.reshape(1, o)
    g2 = ln_gamma.reshape(1, o)
    be2 = ln_beta.reshape(1, o)

    y = None
    for k in range(len(sizes)):
        y = _tc_chunk(y, efeat, gathered[k], wsd, b_in2,
                      wo_t, b_out2, g2, be2, block_e,
                      base_blk=bounds[k] // block_e)
    return y
